# baseline (speedup 1.0000x reference)
import jax
import jax.numpy as jnp
from jax.experimental import pallas as pl
from jax.experimental.pallas import tpu as pltpu

NUM_CLASSES = 64
HIDDEN = 2048
VOCAB = 100000
BATCH = 1024
LANE = 128


def _probe_scatter_kernel(buf_ref, hidden_ref, w_ref, vid_ref, vidv_ref,
                          out_ref, logits_ref, tiles_ref, sem_col):
    del buf_ref
    logits_ref[...] = jax.lax.dot_general(
        hidden_ref[...], w_ref[...],
        dimension_numbers=(((1,), (1,)), ((), ())),
        preferred_element_type=jnp.float32,
    )
    vids = vidv_ref[...]
    for k in range(NUM_CLASSES):
        base_k = (vid_ref[0, k] // LANE) * LANE
        cols = base_k + jax.lax.broadcasted_iota(jnp.int32, (1, LANE), 1)
        hits = cols == vids
        scattered = jax.lax.dot_general(
            logits_ref[...], hits.astype(jnp.float32),
            dimension_numbers=(((1,), (0,)), ((), ())),
            preferred_element_type=jnp.float32,
        )
        covered = jnp.any(hits, axis=0, keepdims=True)
        tiles_ref[k] = jnp.where(covered, scattered, -jnp.inf)

    cols_dmas = []
    for k in range(NUM_CLASSES):
        base_k = pl.multiple_of((vid_ref[0, k] // LANE) * LANE, LANE)
        cols_dmas.append(pltpu.make_async_copy(
            tiles_ref.at[k], out_ref.at[:, pl.ds(base_k, LANE)], sem_col))
    for k, c in enumerate(cols_dmas):
        c.start(priority=k % 2)
    for c in cols_dmas:
        c.wait()


@jax.jit
def kernel(hidden_states, probe_weights, vocab_ids):
    h = hidden_states.astype(jnp.float32)
    vid = vocab_ids.astype(jnp.int32).reshape(1, NUM_CLASSES)
    vidv = vocab_ids.astype(jnp.int32).reshape(NUM_CLASSES, 1)
    buf = jnp.full((BATCH, VOCAB), -jnp.inf, jnp.float32)
    return pl.pallas_call(
        _probe_scatter_kernel,
        grid=(1,),
        in_specs=[
            pl.BlockSpec(memory_space=pl.ANY),
            pl.BlockSpec((BATCH, HIDDEN), lambda i: (0, 0)),
            pl.BlockSpec((NUM_CLASSES, HIDDEN), lambda i: (0, 0)),
            pl.BlockSpec(memory_space=pltpu.SMEM),
            pl.BlockSpec((NUM_CLASSES, 1), lambda i: (0, 0)),
        ],
        out_specs=pl.BlockSpec(memory_space=pl.ANY),
        out_shape=jax.ShapeDtypeStruct((BATCH, VOCAB), jnp.float32),
        input_output_aliases={0: 0},
        scratch_shapes=[
            pltpu.VMEM((BATCH, NUM_CLASSES), jnp.float32),
            pltpu.VMEM((NUM_CLASSES, BATCH, LANE), jnp.float32),
            pltpu.SemaphoreType.DMA,
        ],
        compiler_params=pltpu.CompilerParams(
            dimension_semantics=("arbitrary",),
        ),
    )(buf, h, probe_weights, vid, vidv)
